# baseline (speedup 1.0000x reference)
import functools

import jax
import jax.numpy as jnp
from jax import lax
from jax.experimental import pallas as pl
from jax.experimental.pallas import tpu as pltpu
from jax.experimental.pallas import tpu_sc as plsc

_NUM_CORES = 2
_NUM_SUBCORES = 16
_NUM_WORKERS = _NUM_CORES * _NUM_SUBCORES
_BLK = 256
_SUB = 128


def _make_gather(rows: int, cols: int, dim: int):
    batch = rows * cols
    per_w = batch // _NUM_WORKERS
    n_items = per_w // _BLK
    assert per_w * _NUM_WORKERS == batch and n_items % 2 == 0
    items_per_col = rows // _BLK
    n_pairs = n_items // 2

    mesh = plsc.VectorSubcoreMesh(core_axis_name="c", subcore_axis_name="s")

    scratch = {
        "idx_all": pltpu.VMEM((per_w,), jnp.int32),
        "idx4_all": pltpu.VMEM((per_w,), jnp.int32),
        "rows_a0": pltpu.VMEM((_SUB, 4 * dim), jnp.float32),
        "rows_a1": pltpu.VMEM((_SUB, 4 * dim), jnp.float32),
        "rows_b0": pltpu.VMEM((_SUB, 4 * dim), jnp.float32),
        "rows_b1": pltpu.VMEM((_SUB, 4 * dim), jnp.float32),
        "t_a": pltpu.VMEM((1, dim, _BLK), jnp.float32),
        "t_b": pltpu.VMEM((1, dim, _BLK), jnp.float32),
        "gsem_a": pltpu.SemaphoreType.DMA,
        "gsem_b": pltpu.SemaphoreType.DMA,
        "wsem_a": pltpu.SemaphoreType.DMA,
        "wsem_b": pltpu.SemaphoreType.DMA,
    }

    @functools.partial(
        pl.kernel,
        mesh=mesh,
        out_type=jax.ShapeDtypeStruct((cols, dim, rows), jnp.float32),
        scratch_types=list(scratch.values()),
        compiler_params=pltpu.CompilerParams(
            needs_layout_passes=False, disable_bounds_checks=True
        ),
    )
    def emb(idx_hbm, table_hbm, out_hbm, *scr):
        s = dict(zip(scratch.keys(), scr))
        wid = lax.axis_index("s") * _NUM_CORES + lax.axis_index("c")
        base = wid * per_w
        item0 = wid * n_items
        iota16 = lax.iota(jnp.int32, 16)
        idx_all, idx4_all = s["idx_all"], s["idx4_all"]

        pltpu.sync_copy(idx_hbm.at[pl.ds(base, per_w)], idx_all)

        def shift(k, carry):
            o = pl.multiple_of(k * 16, 16)
            idx4_all[pl.ds(o, 16)] = idx_all[pl.ds(o, 16)] >> 2
            return carry

        lax.fori_loop(0, per_w // 16, shift, 0)

        def coords(g):
            goff = item0 + g
            return goff // items_per_col, (goff % items_per_col) * _BLK

        def fire(g, rows, gsem):
            for j in range(2):
                o = pl.multiple_of(g * _BLK + j * _SUB, _SUB)
                pltpu.async_copy(
                    table_hbm.at[idx4_all.at[pl.ds(o, _SUB)]], rows[j], gsem
                )

        def drain_g(rows, gsem):
            for j in range(2):
                pltpu.make_async_copy(
                    table_hbm.at[idx4_all.at[pl.ds(0, _SUB)]], rows[j], gsem
                ).wait()

        def transpose(g, rows, t_v):
            for half in range(2):

                def group(k, carry, half=half):
                    oo = pl.multiple_of(k * 16, 16)
                    raw = idx_all[pl.ds(g * _BLK + half * _SUB + oo, 16)]
                    cb = (raw & 3) << 5
                    row_ids = oo + iota16
                    for f in range(dim):
                        v = plsc.load_gather(rows[half], [row_ids, cb + f])
                        t_v[0, f, pl.ds(half * _SUB + oo, 16)] = v
                    return carry

                lax.fori_loop(0, _SUB // 16, group, 0)

        def write(g, t_v, wsem):
            col, b0 = coords(g)
            pltpu.async_copy(t_v, out_hbm.at[pl.ds(col, 1), :, pl.ds(b0, _BLK)], wsem)

        def drain_write(t_v, wsem):
            col, b0 = coords(0)
            pltpu.make_async_copy(
                t_v, out_hbm.at[pl.ds(col, 1), :, pl.ds(b0, _BLK)], wsem
            ).wait()

        rows_a = [s["rows_a0"], s["rows_a1"]]
        rows_b = [s["rows_b0"], s["rows_b1"]]

        fire(0, rows_a, s["gsem_a"])

        def pair(q, carry):
            ga = 2 * q
            gb = ga + 1
            fire(gb, rows_b, s["gsem_b"])
            drain_g(rows_a, s["gsem_a"])

            @pl.when(q > 0)
            def _():
                drain_write(s["t_a"], s["wsem_a"])

            transpose(ga, rows_a, s["t_a"])
            write(ga, s["t_a"], s["wsem_a"])

            @pl.when(q < n_pairs - 1)
            def _():
                fire(ga + 2, rows_a, s["gsem_a"])

            drain_g(rows_b, s["gsem_b"])

            @pl.when(q > 0)
            def _():
                drain_write(s["t_b"], s["wsem_b"])

            transpose(gb, rows_b, s["t_b"])
            write(gb, s["t_b"], s["wsem_b"])
            return carry

        lax.fori_loop(0, n_pairs, pair, 0)
        drain_write(s["t_a"], s["wsem_a"])
        drain_write(s["t_b"], s["wsem_b"])

    return emb


def kernel(x, weight):
    rows, cols = x.shape
    vocab, dim = weight.shape
    idx = x.T.reshape(rows * cols).astype(jnp.int32)
    w4 = weight.reshape(vocab // 4, 4 * dim)
    out = _make_gather(rows, cols, dim)(idx, w4)
    return out.transpose(2, 0, 1)


# speedup vs baseline: 1.1988x; 1.1988x over previous
import functools

import jax
import jax.numpy as jnp
from jax import lax
from jax.experimental import pallas as pl
from jax.experimental.pallas import tpu as pltpu
from jax.experimental.pallas import tpu_sc as plsc

_NUM_CORES = 2
_NUM_SUBCORES = 16
_NUM_WORKERS = _NUM_CORES * _NUM_SUBCORES
_BLK = 256
_SUB = 128


def _make_gather(rows: int, cols: int, dim: int):
    batch = rows * cols
    per_w = batch // _NUM_WORKERS
    n_items = per_w // _BLK
    assert per_w * _NUM_WORKERS == batch and n_items % 2 == 0
    items_per_col = rows // _BLK
    n_pairs = n_items // 2

    mesh = plsc.VectorSubcoreMesh(core_axis_name="c", subcore_axis_name="s")

    scratch = {
        "idx_all": pltpu.VMEM((per_w,), jnp.int32),
        "idx4_all": pltpu.VMEM((per_w,), jnp.int32),
        "rows_a0": pltpu.VMEM((_SUB, 4 * dim), jnp.float32),
        "rows_a1": pltpu.VMEM((_SUB, 4 * dim), jnp.float32),
        "rows_b0": pltpu.VMEM((_SUB, 4 * dim), jnp.float32),
        "rows_b1": pltpu.VMEM((_SUB, 4 * dim), jnp.float32),
        "t_a": pltpu.VMEM((1, dim, _BLK), jnp.float32),
        "t_b": pltpu.VMEM((1, dim, _BLK), jnp.float32),
        "gsem_a": pltpu.SemaphoreType.DMA,
        "gsem_b": pltpu.SemaphoreType.DMA,
        "wsem_a": pltpu.SemaphoreType.DMA,
        "wsem_b": pltpu.SemaphoreType.DMA,
    }

    @functools.partial(
        pl.kernel,
        mesh=mesh,
        out_type=jax.ShapeDtypeStruct((cols, dim, rows), jnp.float32),
        scratch_types=list(scratch.values()),
        compiler_params=pltpu.CompilerParams(
            needs_layout_passes=False, disable_bounds_checks=True
        ),
    )
    def emb(idx_hbm, table_hbm, out_hbm, *scr):
        s = dict(zip(scratch.keys(), scr))
        wid = lax.axis_index("s") * _NUM_CORES + lax.axis_index("c")
        base = wid * per_w
        item0 = wid * n_items
        iota16 = lax.iota(jnp.int32, 16)
        idx_all, idx4_all = s["idx_all"], s["idx4_all"]

        pltpu.sync_copy(idx_hbm.at[pl.ds(base, per_w)], idx_all)

        def shift(k, carry):
            o = pl.multiple_of(k * 16, 16)
            idx4_all[pl.ds(o, 16)] = idx_all[pl.ds(o, 16)] >> 2
            return carry

        lax.fori_loop(0, per_w // 16, shift, 0)

        def coords(g):
            goff = item0 + g
            return goff // items_per_col, (goff % items_per_col) * _BLK

        def fire(g, rows, gsem):
            for j in range(2):
                o = pl.multiple_of(g * _BLK + j * _SUB, _SUB)
                pltpu.async_copy(
                    table_hbm.at[idx4_all.at[pl.ds(o, _SUB)]], rows[j], gsem
                )

        def drain_g(rows, gsem):
            for j in range(2):
                pltpu.make_async_copy(
                    table_hbm.at[idx4_all.at[pl.ds(0, _SUB)]], rows[j], gsem
                ).wait()

        def transpose(g, rows, t_v):
            for half in range(2):

                def group(k, carry, half=half):
                    oo = pl.multiple_of(k * 16, 16)
                    raw = idx_all[pl.ds(g * _BLK + half * _SUB + oo, 16)]
                    cb = (raw & 3) << 5
                    row_ids = oo + iota16
                    vs = [
                        plsc.load_gather(rows[half], [row_ids, cb + f])
                        for f in range(dim)
                    ]
                    for f in range(dim):
                        t_v[0, f, pl.ds(half * _SUB + oo, 16)] = vs[f]
                    return carry

                lax.fori_loop(0, _SUB // 16, group, 0)

        def write(g, t_v, wsem):
            col, b0 = coords(g)
            pltpu.async_copy(t_v, out_hbm.at[pl.ds(col, 1), :, pl.ds(b0, _BLK)], wsem)

        def drain_write(t_v, wsem):
            col, b0 = coords(0)
            pltpu.make_async_copy(
                t_v, out_hbm.at[pl.ds(col, 1), :, pl.ds(b0, _BLK)], wsem
            ).wait()

        rows_a = [s["rows_a0"], s["rows_a1"]]
        rows_b = [s["rows_b0"], s["rows_b1"]]

        fire(0, rows_a, s["gsem_a"])

        def pair(q, carry):
            ga = 2 * q
            gb = ga + 1
            fire(gb, rows_b, s["gsem_b"])
            drain_g(rows_a, s["gsem_a"])

            @pl.when(q > 0)
            def _():
                drain_write(s["t_a"], s["wsem_a"])

            transpose(ga, rows_a, s["t_a"])
            write(ga, s["t_a"], s["wsem_a"])

            @pl.when(q < n_pairs - 1)
            def _():
                fire(ga + 2, rows_a, s["gsem_a"])

            drain_g(rows_b, s["gsem_b"])

            @pl.when(q > 0)
            def _():
                drain_write(s["t_b"], s["wsem_b"])

            transpose(gb, rows_b, s["t_b"])
            write(gb, s["t_b"], s["wsem_b"])
            return carry

        lax.fori_loop(0, n_pairs, pair, 0)
        drain_write(s["t_a"], s["wsem_a"])
        drain_write(s["t_b"], s["wsem_b"])

    return emb


def kernel(x, weight):
    rows, cols = x.shape
    vocab, dim = weight.shape
    idx = x.T.reshape(rows * cols).astype(jnp.int32)
    w4 = weight.reshape(vocab // 4, 4 * dim)
    out = _make_gather(rows, cols, dim)(idx, w4)
    return out.transpose(2, 0, 1)
